# baseline (speedup 1.0000x reference)
import jax
import jax.numpy as jnp
from jax.experimental import pallas as pl
from jax.experimental.pallas import tpu as pltpu

_N = 4096
_R = 512
_G = _N // _R
_RC = 128
_TC = _N // _RC
_RP = 1024
_RS = 512
_T = _TC + 7 * (_N // _RP) + _N // _RS


def _leaky(z):
    return jnp.where(z >= 0, z, 0.2 * z)


def _dot(a, b):
    return jnp.dot(a, b, preferred_element_type=jnp.float32)


def _soft_assign(z, cluster):
    zn = jnp.sum(z * z, axis=1, keepdims=True)
    cn = jnp.sum(cluster * cluster, axis=1)[None, :]
    d2 = zn + cn - 2.0 * _dot(z, cluster.T)
    q = 1.0 / (1.0 + d2)
    return q / jnp.sum(q, axis=1, keepdims=True)


def _mega_kernel(adj_ref, x_ref, ew0, eb0, ew1, eb1, ew2, eb2, ew3, eb3,
                 gw0, gw1, gw2, cl_ref, a_ref, gamma_ref,
                 zae_out, q1_out, zs_out, zt_out, t3_out,
                 adjv, va, vb):
    t = pl.program_id(0)

    @pl.when(t < _TC)
    def _():
        cs = pl.ds(t * _RC, _RC)
        adjv[cs, :] = adj_ref[...].astype(jnp.bfloat16)
        x = x_ref[...]
        z = _leaky(_dot(x, ew0[...]) + eb0[...])
        z = _leaky(_dot(z, ew1[...]) + eb1[...])
        z = _leaky(_dot(z, ew2[...]) + eb2[...])
        zae = _dot(z, ew3[...]) + eb3[...]
        zae_out[cs, :] = zae
        q1_out[cs, :] = _soft_assign(zae, cl_ref[...])
        wg = _dot(_dot(gw0[...], gw1[...]), gw2[...])
        va[cs, :] = _dot(x, wg)

    def _pass(lo, dst, src):
        @pl.when(jnp.logical_and(t >= lo, t < lo + _N // _RP))
        def _():
            rr = pl.ds((t - lo) * _RP, _RP)
            dst[rr, :] = _dot(adjv[rr, :], src[...].astype(jnp.bfloat16))

    _pass(_TC, vb, va)
    _pass(_TC + 4, va, vb)

    @pl.when(jnp.logical_and(t >= _TC + 8, t < _TC + 12))
    def _():
        rr = pl.ds((t - (_TC + 8)) * _RP, _RP)
        zs_r = _dot(adjv[rr, :], va[...].astype(jnp.bfloat16))
        zs_out[rr, :] = zs_r
        a_r = a_ref[rr, :]
        vb[rr, :] = a_r * zae_out[rr, :] + (1.0 - a_r) * zs_r

    _pass(_TC + 12, va, vb)

    @pl.when(jnp.logical_and(t >= _TC + 16, t < _TC + 24))
    def _():
        rs = pl.ds((t - (_TC + 16)) * _RS, _RS)
        zl_r = va[rs, :]
        m = jnp.full((_RS, 1), -jnp.inf, dtype=jnp.float32)
        den = jnp.zeros((_RS, 1), dtype=jnp.float32)
        acc = jnp.zeros((_RS, 32), dtype=jnp.float32)
        for c in range(16):
            zl_c = va[pl.ds(c * (_N // 16), _N // 16), :]
            sc = _dot(zl_r, zl_c.T)
            m_new = jnp.maximum(m, jnp.max(sc, axis=1, keepdims=True))
            alpha = jnp.exp(m - m_new)
            pch = jnp.exp(sc - m_new)
            den = den * alpha + jnp.sum(pch, axis=1, keepdims=True)
            acc = acc * alpha + _dot(pch, zl_c)
            m = m_new
        zt_r = gamma_ref[0, 0] * (acc / den) + zl_r
        zt_out[rs, :] = zt_r
        vb[rs, :] = zt_r

    _pass(_TC + 24, va, vb)
    _pass(_TC + 28, vb, va)

    @pl.when(t >= _TC + 32)
    def _():
        rr = pl.ds((t - (_TC + 32)) * _RP, _RP)
        t3_out[rr, :] = _dot(adjv[rr, :], vb[...].astype(jnp.bfloat16))


def _tail_kernel(zt_ref, t3_ref, zs_ref, zsf_ref, t3f_ref,
                 dw0, db0, dw1, db1, dw2, db2, dw3, db3,
                 gw0, gw1, gw2, cl,
                 xhat_out, zhat_out, q_out, q2_out, ah_out):
    zt = zt_ref[...]
    d = _leaky(_dot(zt, dw0[...]) + db0[...])
    d = _leaky(_dot(d, dw1[...]) + db1[...])
    d = _leaky(_dot(d, dw2[...]) + db2[...])
    xhat_out[...] = _dot(d, dw3[...]) + db3[...]
    ug = _dot(_dot(gw0[...], gw1[...]), gw2[...])
    t3 = t3_ref[...]
    zhat_out[...] = _dot(t3, ug)
    q_out[...] = _soft_assign(zt, cl[...])
    q2_out[...] = _soft_assign(zs_ref[...], cl[...])
    sc52 = jnp.float32(2.0 ** -52)
    t3_s = t3 * sc52
    tp_s = _dot(t3_s, _dot(ug, ug.T))
    a2s = _dot(tp_s, (t3f_ref[...] * sc52).T)
    a1 = _dot(zs_ref[...], zsf_ref[...].T)
    ah_out[...] = 1.0 + 0.5 * (jnp.tanh(0.5 * a1)
                               + jnp.tanh(a2s * jnp.float32(2.0 ** 103)))


def _full(arr):
    nd = arr.ndim
    return pl.BlockSpec(arr.shape, lambda i, _n=nd: (0,) * _n)


def _row(last):
    return pl.BlockSpec((_R, last), lambda i: (i, 0))


def _sds(shape):
    return jax.ShapeDtypeStruct(shape, jnp.float32)


def kernel(x, adj, params):
    p = params
    b = {k: p[k].reshape(1, -1) for k in p if k.startswith('ae_') and '_b' in k}
    gamma = p['gamma'].reshape(1, 1)
    cl = p['cluster']

    def cfull(shape):
        return pl.BlockSpec(shape, lambda t_: (0,) * len(shape))

    adj_spec = pl.BlockSpec(
        (_RC, _N), lambda t_: (jnp.where(t_ < _TC, t_, _TC - 1), 0))
    x_spec = pl.BlockSpec(
        (_RC, 512), lambda t_: (jnp.where(t_ < _TC, t_, _TC - 1), 0))

    mega_ins = [p['ae_enc_w0'], b['ae_enc_b0'], p['ae_enc_w1'], b['ae_enc_b1'],
                p['ae_enc_w2'], b['ae_enc_b2'], p['ae_enc_w3'], b['ae_enc_b3'],
                p['gae_enc_w0'], p['gae_enc_w1'], p['gae_enc_w2'],
                cl, p['a'], gamma]
    zae, q1, zs, zt, t3 = pl.pallas_call(
        _mega_kernel,
        grid=(_T,),
        in_specs=[adj_spec, x_spec] + [cfull(v.shape) for v in mega_ins],
        out_specs=[cfull((_N, 32)), cfull((_N, 10)), cfull((_N, 32)),
                   cfull((_N, 32)), cfull((_N, 32))],
        out_shape=[_sds((_N, 32)), _sds((_N, 10)), _sds((_N, 32)),
                   _sds((_N, 32)), _sds((_N, 32))],
        scratch_shapes=[pltpu.VMEM((_N, _N), jnp.bfloat16),
                        pltpu.VMEM((_N, 32), jnp.float32),
                        pltpu.VMEM((_N, 32), jnp.float32)],
    )(adj, x, *mega_ins)

    xhat, zhat, q, q2, adj_hat = pl.pallas_call(
        _tail_kernel,
        grid=(_G,),
        in_specs=[_row(32), _row(32), _row(32), _full(zs), _full(t3),
                  _full(p['ae_dec_w0']), _full(b['ae_dec_b0']),
                  _full(p['ae_dec_w1']), _full(b['ae_dec_b1']),
                  _full(p['ae_dec_w2']), _full(b['ae_dec_b2']),
                  _full(p['ae_dec_w3']), _full(b['ae_dec_b3']),
                  _full(p['gae_dec_w0']), _full(p['gae_dec_w1']),
                  _full(p['gae_dec_w2']), _full(cl)],
        out_specs=[_row(512), _row(512), _row(10), _row(10), _row(_N)],
        out_shape=[_sds((_N, 512)), _sds((_N, 512)), _sds((_N, 10)),
                   _sds((_N, 10)), _sds((_N, _N))],
    )(zt, t3, zs, zs, t3,
      p['ae_dec_w0'], b['ae_dec_b0'], p['ae_dec_w1'], b['ae_dec_b1'],
      p['ae_dec_w2'], b['ae_dec_b2'], p['ae_dec_w3'], b['ae_dec_b3'],
      p['gae_dec_w0'], p['gae_dec_w1'], p['gae_dec_w2'], cl)

    return (xhat, zhat, adj_hat, zae, zs, q, q1, q2, zt)


# speedup vs baseline: 1.0246x; 1.0246x over previous
import jax
import jax.numpy as jnp
from jax.experimental import pallas as pl
from jax.experimental.pallas import tpu as pltpu

_N = 4096
_R = 512
_G = _N // _R
_RC = 128
_TC = _N // _RC
_RP = 1024
_RS = 512
_T = _TC + 7 * (_N // _RP) + _N // _RS


def _leaky(z):
    return jnp.where(z >= 0, z, 0.2 * z)


def _dot(a, b):
    return jnp.dot(a, b, preferred_element_type=jnp.float32)


def _soft_assign(z, cluster):
    zn = jnp.sum(z * z, axis=1, keepdims=True)
    cn = jnp.sum(cluster * cluster, axis=1)[None, :]
    d2 = zn + cn - 2.0 * _dot(z, cluster.T)
    q = 1.0 / (1.0 + d2)
    return q / jnp.sum(q, axis=1, keepdims=True)


def _pre_kernel(x_ref, w0, b0, w1, b1, w2, b2, w3, b3,
                gw0, gw1, gw2, cl, zae_out, q1_out, v0_out):
    x = x_ref[...]
    z = _leaky(_dot(x, w0[...]) + b0[...])
    z = _leaky(_dot(z, w1[...]) + b1[...])
    z = _leaky(_dot(z, w2[...]) + b2[...])
    zae = _dot(z, w3[...]) + b3[...]
    zae_out[...] = zae
    q1_out[...] = _soft_assign(zae, cl[...])
    wg = _dot(_dot(gw0[...], gw1[...]), gw2[...])
    v0_out[...] = _dot(x, wg)


def _mega_kernel(adj_ref, v0_ref, zae_ref, a_ref, gamma_ref,
                 zs_out, zt_out, t3_out,
                 adjv, va, vb):
    t = pl.program_id(0)

    @pl.when(t < _TC)
    def _():
        cs = pl.ds(t * _RC, _RC)
        adjv[cs, :] = adj_ref[...].astype(jnp.bfloat16)
        va[cs, :] = v0_ref[cs, :]

    def _pass(lo, dst, src):
        @pl.when(jnp.logical_and(t >= lo, t < lo + _N // _RP))
        def _():
            rr = pl.ds((t - lo) * _RP, _RP)
            dst[rr, :] = _dot(adjv[rr, :], src[...].astype(jnp.bfloat16))

    _pass(_TC, vb, va)
    _pass(_TC + 4, va, vb)

    @pl.when(jnp.logical_and(t >= _TC + 8, t < _TC + 12))
    def _():
        rr = pl.ds((t - (_TC + 8)) * _RP, _RP)
        zs_r = _dot(adjv[rr, :], va[...].astype(jnp.bfloat16))
        zs_out[rr, :] = zs_r
        a_r = a_ref[rr, :]
        vb[rr, :] = a_r * zae_ref[rr, :] + (1.0 - a_r) * zs_r

    _pass(_TC + 12, va, vb)

    @pl.when(jnp.logical_and(t >= _TC + 16, t < _TC + 24))
    def _():
        rs = pl.ds((t - (_TC + 16)) * _RS, _RS)
        zl_r = va[rs, :]
        m = jnp.full((_RS, 1), -jnp.inf, dtype=jnp.float32)
        den = jnp.zeros((_RS, 1), dtype=jnp.float32)
        acc = jnp.zeros((_RS, 32), dtype=jnp.float32)
        for c in range(16):
            zl_c = va[pl.ds(c * (_N // 16), _N // 16), :]
            sc = _dot(zl_r, zl_c.T)
            m_new = jnp.maximum(m, jnp.max(sc, axis=1, keepdims=True))
            alpha = jnp.exp(m - m_new)
            pch = jnp.exp(sc - m_new)
            den = den * alpha + jnp.sum(pch, axis=1, keepdims=True)
            acc = acc * alpha + _dot(pch, zl_c)
            m = m_new
        zt_r = gamma_ref[0, 0] * (acc / den) + zl_r
        zt_out[rs, :] = zt_r
        vb[rs, :] = zt_r

    _pass(_TC + 24, va, vb)
    _pass(_TC + 28, vb, va)

    @pl.when(t >= _TC + 32)
    def _():
        rr = pl.ds((t - (_TC + 32)) * _RP, _RP)
        t3_out[rr, :] = _dot(adjv[rr, :], vb[...].astype(jnp.bfloat16))


def _tail_kernel(zt_ref, t3_ref, zs_ref, zsf_ref, t3f_ref,
                 dw0, db0, dw1, db1, dw2, db2, dw3, db3,
                 gw0, gw1, gw2, cl,
                 xhat_out, zhat_out, q_out, q2_out, ah_out):
    zt = zt_ref[...]
    d = _leaky(_dot(zt, dw0[...]) + db0[...])
    d = _leaky(_dot(d, dw1[...]) + db1[...])
    d = _leaky(_dot(d, dw2[...]) + db2[...])
    xhat_out[...] = _dot(d, dw3[...]) + db3[...]
    ug = _dot(_dot(gw0[...], gw1[...]), gw2[...])
    t3 = t3_ref[...]
    zhat_out[...] = _dot(t3, ug)
    q_out[...] = _soft_assign(zt, cl[...])
    q2_out[...] = _soft_assign(zs_ref[...], cl[...])
    sc52 = jnp.float32(2.0 ** -52)
    t3_s = t3 * sc52
    tp_s = _dot(t3_s, _dot(ug, ug.T))
    a2s = _dot(tp_s, (t3f_ref[...] * sc52).T)
    a1 = _dot(zs_ref[...], zsf_ref[...].T)
    ah_out[...] = 1.0 + 0.5 * (jnp.tanh(0.5 * a1)
                               + jnp.tanh(a2s * jnp.float32(2.0 ** 103)))


def _full(arr):
    nd = arr.ndim
    return pl.BlockSpec(arr.shape, lambda i, _n=nd: (0,) * _n)


def _row(last):
    return pl.BlockSpec((_R, last), lambda i: (i, 0))


def _sds(shape):
    return jax.ShapeDtypeStruct(shape, jnp.float32)


def kernel(x, adj, params):
    p = params
    b = {k: p[k].reshape(1, -1) for k in p if k.startswith('ae_') and '_b' in k}
    gamma = p['gamma'].reshape(1, 1)
    cl = p['cluster']

    zae, q1, v0 = pl.pallas_call(
        _pre_kernel,
        grid=(_G,),
        in_specs=[_row(512),
                  _full(p['ae_enc_w0']), _full(b['ae_enc_b0']),
                  _full(p['ae_enc_w1']), _full(b['ae_enc_b1']),
                  _full(p['ae_enc_w2']), _full(b['ae_enc_b2']),
                  _full(p['ae_enc_w3']), _full(b['ae_enc_b3']),
                  _full(p['gae_enc_w0']), _full(p['gae_enc_w1']),
                  _full(p['gae_enc_w2']), _full(cl)],
        out_specs=[_row(32), _row(10), _row(32)],
        out_shape=[_sds((_N, 32)), _sds((_N, 10)), _sds((_N, 32))],
    )(x, p['ae_enc_w0'], b['ae_enc_b0'], p['ae_enc_w1'], b['ae_enc_b1'],
      p['ae_enc_w2'], b['ae_enc_b2'], p['ae_enc_w3'], b['ae_enc_b3'],
      p['gae_enc_w0'], p['gae_enc_w1'], p['gae_enc_w2'], cl)

    def cfull(shape):
        return pl.BlockSpec(shape, lambda t_: (0,) * len(shape))

    adj_spec = pl.BlockSpec(
        (_RC, _N), lambda t_: (jnp.where(t_ < _TC, t_, _TC - 1), 0))

    mega_ins = [v0, zae, p['a'], gamma]
    zs, zt, t3 = pl.pallas_call(
        _mega_kernel,
        grid=(_T,),
        in_specs=[adj_spec] + [cfull(v.shape) for v in mega_ins],
        out_specs=[cfull((_N, 32)), cfull((_N, 32)), cfull((_N, 32))],
        out_shape=[_sds((_N, 32)), _sds((_N, 32)), _sds((_N, 32))],
        scratch_shapes=[pltpu.VMEM((_N, _N), jnp.bfloat16),
                        pltpu.VMEM((_N, 32), jnp.float32),
                        pltpu.VMEM((_N, 32), jnp.float32)],
    )(adj, *mega_ins)

    xhat, zhat, q, q2, adj_hat = pl.pallas_call(
        _tail_kernel,
        grid=(_G,),
        in_specs=[_row(32), _row(32), _row(32), _full(zs), _full(t3),
                  _full(p['ae_dec_w0']), _full(b['ae_dec_b0']),
                  _full(p['ae_dec_w1']), _full(b['ae_dec_b1']),
                  _full(p['ae_dec_w2']), _full(b['ae_dec_b2']),
                  _full(p['ae_dec_w3']), _full(b['ae_dec_b3']),
                  _full(p['gae_dec_w0']), _full(p['gae_dec_w1']),
                  _full(p['gae_dec_w2']), _full(cl)],
        out_specs=[_row(512), _row(512), _row(10), _row(10), _row(_N)],
        out_shape=[_sds((_N, 512)), _sds((_N, 512)), _sds((_N, 10)),
                   _sds((_N, 10)), _sds((_N, _N))],
    )(zt, t3, zs, zs, t3,
      p['ae_dec_w0'], b['ae_dec_b0'], p['ae_dec_w1'], b['ae_dec_b1'],
      p['ae_dec_w2'], b['ae_dec_b2'], p['ae_dec_w3'], b['ae_dec_b3'],
      p['gae_dec_w0'], p['gae_dec_w1'], p['gae_dec_w2'], cl)

    return (xhat, zhat, adj_hat, zae, zs, q, q1, q2, zt)
